# baseline (speedup 1.0000x reference)
import functools

import numpy as np
import jax
import jax.numpy as jnp
from jax import lax
from jax.experimental import pallas as pl
from jax.experimental.pallas import tpu as pltpu
from jax.experimental.pallas import tpu_sc as plsc

B, C = 16384, 1024
N_ANCHORS, PER_ANCHOR, BODY_LEN = 8, 4, 16
NCONS = N_ANCHORS * PER_ANCHOR
DP = 128
ROWS = 2048
GRID = B // ROWS


def _constraint_list():
    rng = np.random.RandomState(0)
    cons = []
    for a in range(N_ANCHORS):
        for _ in range(PER_ANCHOR):
            var_ids = [a] + [int(v) for v in rng.randint(0, C, size=BODY_LEN - 1)]
            coeffs = [float(c) for c in rng.uniform(0.5, 1.5, size=BODY_LEN)]
            signs = [bool(s) for s in rng.randint(0, 2, size=BODY_LEN)]
            candidates = [i for i in range(BODY_LEN) if i != a]
            mask_index = candidates[int(rng.randint(0, len(candidates)))]
            cons.append((a, var_ids, coeffs, signs, var_ids[mask_index]))
    return cons


_CONS = _constraint_list()
_P_COLS = list(range(N_ANCHORS)) + sorted({m for (_, _, _, _, m) in _CONS
                                           if m >= N_ANCHORS})
_NB = len(_P_COLS)
_GOFF = _NB
_COL2B = {c: i for i, c in enumerate(_P_COLS)}

_MP = np.zeros((C, DP), np.float32)
for _i, _c in enumerate(_P_COLS):
    _MP[_c, _i] = 1.0
_MG = np.zeros((C, DP), np.float32)
for _j, (_a, _vids, _cf, _sg, _m) in enumerate(_CONS):
    for _i in range(BODY_LEN):
        if _vids[_i] != _m:
            _MG[_vids[_i], _GOFF + _j] += _cf[_i] * (-1.0 if _sg[_i] else 1.0)
_CB = np.zeros((NCONS, DP), np.float32)
for _j, (_a, _vids, _cf, _sg, _m) in enumerate(_CONS):
    _CB[_j, _GOFF + _j] = 1.0
    if _m >= N_ANCHORS:
        _CB[_j, _COL2B[_m]] = 1.0
_A0 = np.zeros((N_ANCHORS, DP), np.float32)
for _a in range(N_ANCHORS):
    _A0[_a, _a] = 1.0

_HI = jax.lax.Precision.HIGHEST

_SC_NC, _SC_NS = 2, 16
_SC_W = _SC_NC * _SC_NS
_SC_ROWS = B // _SC_W


@functools.partial(
    pl.kernel,
    out_type=jax.ShapeDtypeStruct((B, C), jnp.float32),
    mesh=plsc.VectorSubcoreMesh(core_axis_name="c", subcore_axis_name="s"),
    compiler_params=pltpu.CompilerParams(use_tc_tiling_on_sc=True),
)
def _sc_copy(preds_hbm, out_hbm):
    wid = lax.axis_index("s") * _SC_NC + lax.axis_index("c")
    base = wid * _SC_ROWS
    pltpu.sync_copy(preds_hbm.at[pl.ds(base, _SC_ROWS)],
                    out_hbm.at[pl.ds(base, _SC_ROWS)])


def _decide(gram, a0, cb):
    acc = a0
    rowid = jax.lax.broadcasted_iota(jnp.int32, (N_ANCHORS, DP), 0)
    for j, (a, _vids, _cf, _sg, m) in enumerate(_CONS):
        cbj = cb[j:j + 1, :]
        if m < N_ANCHORS:
            alpha_c = acc[m:m + 1, :] + cbj
        else:
            alpha_c = cbj
        pa = acc[a:a + 1, :]
        qc = jnp.sum(jnp.dot(alpha_c, gram, precision=_HI) * alpha_c)
        qp = jnp.sum(jnp.dot(pa, gram, precision=_HI) * pa)
        new_row = jnp.where(qc > qp, alpha_c, pa)
        acc = jnp.where(rowid == a, new_row, acc)
    return jnp.concatenate(
        [acc, jnp.zeros((DP - N_ANCHORS, DP), acc.dtype)], axis=0)


def _k1_body(preds_ref, gt_ref, mp_ref, mg_ref, a0_ref, cb_ref,
             basis_ref, gram_ref, ap_ref):
    basis = (jnp.dot(preds_ref[...], mp_ref[...]) +
             jnp.dot(gt_ref[...], mg_ref[...]))
    basis_ref[...] = basis.astype(jnp.bfloat16)

    @pl.when(pl.program_id(0) == 0)
    def _init():
        gram_ref[...] = jnp.zeros_like(gram_ref)

    gram_ref[...] += jax.lax.dot_general(
        basis, basis, (((0,), (0,)), ((), ())))

    @pl.when(pl.program_id(0) == GRID - 1)
    def _final():
        ap_ref[...] = _decide(gram_ref[...], a0_ref[...], cb_ref[...])


def _k2_body(preds_ref, basis_ref, ap_ref, sc_ref, out_ref):
    del sc_ref
    cols = jax.lax.dot_general(
        basis_ref[...].astype(jnp.float32), ap_ref[...],
        (((1,), (1,)), ((), ())),
        precision=_HI)
    lane = jax.lax.broadcasted_iota(jnp.int32, (ROWS, DP), 1)
    out_ref[...] = jnp.where(lane < N_ANCHORS, cols, preds_ref[...])


def kernel(preds, ground_truth):
    mp = jnp.asarray(_MP)
    mg = jnp.asarray(_MG)
    a0 = jnp.asarray(_A0)
    cb = jnp.asarray(_CB)

    sc_pass = _sc_copy(preds)

    basis, _gram, ap = pl.pallas_call(
        _k1_body,
        grid=(GRID,),
        in_specs=[
            pl.BlockSpec((ROWS, C), lambda i: (i, 0)),
            pl.BlockSpec((ROWS, C), lambda i: (i, 0)),
            pl.BlockSpec((C, DP), lambda i: (0, 0)),
            pl.BlockSpec((C, DP), lambda i: (0, 0)),
            pl.BlockSpec((N_ANCHORS, DP), lambda i: (0, 0)),
            pl.BlockSpec((NCONS, DP), lambda i: (0, 0)),
        ],
        out_specs=[
            pl.BlockSpec((ROWS, DP), lambda i: (i, 0)),
            pl.BlockSpec((DP, DP), lambda i: (0, 0)),
            pl.BlockSpec((DP, DP), lambda i: (0, 0)),
        ],
        out_shape=[
            jax.ShapeDtypeStruct((B, DP), jnp.bfloat16),
            jax.ShapeDtypeStruct((DP, DP), jnp.float32),
            jax.ShapeDtypeStruct((DP, DP), jnp.float32),
        ],
    )(preds, ground_truth, mp, mg, a0, cb)

    out = pl.pallas_call(
        _k2_body,
        grid=(GRID,),
        in_specs=[
            pl.BlockSpec((ROWS, DP), lambda i: (i, 0)),
            pl.BlockSpec((ROWS, DP), lambda i: (i, 0)),
            pl.BlockSpec((DP, DP), lambda i: (0, 0)),
            pl.BlockSpec(memory_space=pl.ANY),
        ],
        out_specs=pl.BlockSpec((ROWS, DP), lambda i: (i, 0)),
        out_shape=jax.ShapeDtypeStruct((B, C), jnp.float32),
        input_output_aliases={3: 0},
    )(preds, basis, ap, sc_pass)
    return out


# speedup vs baseline: 23.3748x; 23.3748x over previous
import numpy as np
import jax
import jax.numpy as jnp
from jax.experimental import pallas as pl
from jax.experimental.pallas import tpu as pltpu

B, C = 16384, 1024
N_ANCHORS, PER_ANCHOR, BODY_LEN = 8, 4, 16
NCONS = N_ANCHORS * PER_ANCHOR
DP = 128
ROWS = 2048
GRID = B // ROWS


def _constraint_list():
    rng = np.random.RandomState(0)
    cons = []
    for a in range(N_ANCHORS):
        for _ in range(PER_ANCHOR):
            var_ids = [a] + [int(v) for v in rng.randint(0, C, size=BODY_LEN - 1)]
            coeffs = [float(c) for c in rng.uniform(0.5, 1.5, size=BODY_LEN)]
            signs = [bool(s) for s in rng.randint(0, 2, size=BODY_LEN)]
            candidates = [i for i in range(BODY_LEN) if i != a]
            mask_index = candidates[int(rng.randint(0, len(candidates)))]
            cons.append((a, var_ids, coeffs, signs, var_ids[mask_index]))
    return cons


_CONS = _constraint_list()
_P_COLS = list(range(N_ANCHORS)) + sorted({m for (_, _, _, _, m) in _CONS
                                           if m >= N_ANCHORS})
_NB = len(_P_COLS)
_GOFF = _NB
_COL2B = {c: i for i, c in enumerate(_P_COLS)}

_MP = np.zeros((C, DP), np.float32)
for _i, _c in enumerate(_P_COLS):
    _MP[_c, _i] = 1.0
_MG = np.zeros((C, DP), np.float32)
for _j, (_a, _vids, _cf, _sg, _m) in enumerate(_CONS):
    for _i in range(BODY_LEN):
        if _vids[_i] != _m:
            _MG[_vids[_i], _GOFF + _j] += _cf[_i] * (-1.0 if _sg[_i] else 1.0)
_CB = np.zeros((NCONS, DP), np.float32)
for _j, (_a, _vids, _cf, _sg, _m) in enumerate(_CONS):
    _CB[_j, _GOFF + _j] = 1.0
    if _m >= N_ANCHORS:
        _CB[_j, _COL2B[_m]] = 1.0
_A0 = np.zeros((N_ANCHORS, DP), np.float32)
for _a in range(N_ANCHORS):
    _A0[_a, _a] = 1.0

_HI = jax.lax.Precision.HIGHEST


def _decide(gram, a0, cb):
    acc = a0
    rowid = jax.lax.broadcasted_iota(jnp.int32, (N_ANCHORS, DP), 0)
    for j, (a, _vids, _cf, _sg, m) in enumerate(_CONS):
        cbj = cb[j:j + 1, :]
        if m < N_ANCHORS:
            alpha_c = acc[m:m + 1, :] + cbj
        else:
            alpha_c = cbj
        pa = acc[a:a + 1, :]
        qc = jnp.sum(jnp.dot(alpha_c, gram, precision=_HI) * alpha_c)
        qp = jnp.sum(jnp.dot(pa, gram, precision=_HI) * pa)
        new_row = jnp.where(qc > qp, alpha_c, pa)
        acc = jnp.where(rowid == a, new_row, acc)
    return jnp.concatenate(
        [acc, jnp.zeros((DP - N_ANCHORS, DP), acc.dtype)], axis=0)


def _k1_body(preds_ref, gt_ref, mp_ref, mg_ref, a0_ref, cb_ref,
             pass_ref, basis_ref, gram_ref, ap_ref):
    preds = preds_ref[...]
    pass_ref[...] = preds
    basis = jnp.dot(preds, mp_ref[...]) + jnp.dot(gt_ref[...], mg_ref[...])
    basis_ref[...] = basis.astype(jnp.bfloat16)

    @pl.when(pl.program_id(0) == 0)
    def _init():
        gram_ref[...] = jnp.zeros_like(gram_ref)

    gram_ref[...] += jax.lax.dot_general(
        basis, basis, (((0,), (0,)), ((), ())))

    @pl.when(pl.program_id(0) == GRID - 1)
    def _final():
        ap_ref[...] = _decide(gram_ref[...], a0_ref[...], cb_ref[...])


def _k2_body(pass_ref, basis_ref, ap_ref, out_ref):
    cols = jax.lax.dot_general(
        basis_ref[...].astype(jnp.float32), ap_ref[...],
        (((1,), (1,)), ((), ())),
        precision=_HI)
    lane = jax.lax.broadcasted_iota(jnp.int32, (ROWS, DP), 1)
    out_ref[...] = jnp.where(lane < N_ANCHORS, cols, pass_ref[...])


def kernel(preds, ground_truth):
    mp = jnp.asarray(_MP)
    mg = jnp.asarray(_MG)
    a0 = jnp.asarray(_A0)
    cb = jnp.asarray(_CB)

    passthrough, basis, _gram, ap = pl.pallas_call(
        _k1_body,
        grid=(GRID,),
        in_specs=[
            pl.BlockSpec((ROWS, C), lambda i: (i, 0)),
            pl.BlockSpec((ROWS, C), lambda i: (i, 0)),
            pl.BlockSpec((C, DP), lambda i: (0, 0)),
            pl.BlockSpec((C, DP), lambda i: (0, 0)),
            pl.BlockSpec((N_ANCHORS, DP), lambda i: (0, 0)),
            pl.BlockSpec((NCONS, DP), lambda i: (0, 0)),
        ],
        out_specs=[
            pl.BlockSpec((ROWS, C), lambda i: (i, 0)),
            pl.BlockSpec((ROWS, DP), lambda i: (i, 0)),
            pl.BlockSpec((DP, DP), lambda i: (0, 0)),
            pl.BlockSpec((DP, DP), lambda i: (0, 0)),
        ],
        out_shape=[
            jax.ShapeDtypeStruct((B, C), jnp.float32),
            jax.ShapeDtypeStruct((B, DP), jnp.bfloat16),
            jax.ShapeDtypeStruct((DP, DP), jnp.float32),
            jax.ShapeDtypeStruct((DP, DP), jnp.float32),
        ],
    )(preds, ground_truth, mp, mg, a0, cb)

    out = pl.pallas_call(
        _k2_body,
        grid=(GRID,),
        in_specs=[
            pl.BlockSpec((ROWS, DP), lambda i: (i, 0)),
            pl.BlockSpec((ROWS, DP), lambda i: (i, 0)),
            pl.BlockSpec((DP, DP), lambda i: (0, 0)),
        ],
        out_specs=pl.BlockSpec((ROWS, DP), lambda i: (i, 0)),
        out_shape=jax.ShapeDtypeStruct((B, C), jnp.float32),
        input_output_aliases={0: 0},
    )(passthrough, basis, ap)
    return out
